# baseline (speedup 1.0000x reference)
import functools

import jax
import jax.numpy as jnp
from jax import lax
from jax.experimental import pallas as pl
from jax.experimental.pallas import tpu as pltpu
from jax.experimental.pallas import tpu_sc as plsc

_N_KEYS = 100000
_BITS = 512
_LANES = 16
_NVEC = _BITS // _LANES

_CHUNK = 80
_NCHUNKS = _N_KEYS // _CHUNK
_NW = 32
_CHUNKS_PER_W = -(-_NCHUNKS // _NW)


_GATHER_DNUMS = lax.GatherDimensionNumbers(
    offset_dims=(), collapsed_slice_dims=(0,), start_index_map=(0,))


def _lane_permute(v, idx):
    return lax.gather(
        v, idx[:, None], dimension_numbers=_GATHER_DNUMS, slice_sizes=(1,),
        mode=lax.GatherScatterMode.PROMISE_IN_BOUNDS)


def _lane_sum_all(v, perms):
    for p in perms:
        v = v + _lane_permute(v, p)
    return v


def _make_sc_kernel():
    mesh = plsc.VectorSubcoreMesh(core_axis_name="c", subcore_axis_name="s")

    @functools.partial(
        pl.kernel,
        mesh=mesh,
        out_type=jax.ShapeDtypeStruct((_N_KEYS,), jnp.float32),
        scratch_types=[
            pltpu.VMEM((_BITS,), jnp.float32),
            pltpu.VMEM((_CHUNK * _BITS,), jnp.float32),
            pltpu.VMEM((_CHUNK,), jnp.float32),
        ],
    )
    def sc_kernel(q_hbm, keys_hbm, out_hbm, q_v, k_v, o_v):
        cid = lax.axis_index("c")
        sid = lax.axis_index("s")
        wid = sid * 2 + cid

        pltpu.sync_copy(q_hbm, q_v)

        s_regs = []
        n1_acc = jnp.zeros((_LANES,), jnp.float32)
        for i in range(_NVEC):
            qc = q_v[pl.ds(i * _LANES, _LANES)]
            s_regs.append(1.0 - 2.0 * qc)
            n1_acc = n1_acc + qc
        lane_iota = lax.iota(jnp.int32, _LANES)
        perms = [lane_iota ^ (1 << k) for k in range(4)]
        thresh = 1.0 - _lane_sum_all(n1_acc, perms)

        def chunk_body(j, carry):
            cidx = wid + j * _NW

            @pl.when(cidx < _NCHUNKS)
            def _():
                base = cidx * _CHUNK * _BITS
                pltpu.sync_copy(keys_hbm.at[pl.ds(base, _CHUNK * _BITS)], k_v)

                def group_body(g, gcarry):
                    goff = g * (_LANES * _BITS)
                    res = jnp.zeros((_LANES,), jnp.float32)
                    for r in range(_LANES):
                        off = goff + r * _BITS
                        accs = [jnp.zeros((_LANES,), jnp.float32)
                                for _ in range(4)]
                        for i in range(_NVEC):
                            kk = k_v[pl.ds(off + i * _LANES, _LANES)]
                            accs[i % 4] = accs[i % 4] + kk * s_regs[i]
                        acc = (accs[0] + accs[1]) + (accs[2] + accs[3])
                        d = _lane_sum_all(acc, perms)
                        res = jnp.where(lane_iota == r, d, res)
                    res = jnp.where(res <= thresh, 1.0, 0.0).astype(jnp.float32)
                    o_v[pl.ds(g * _LANES, _LANES)] = res
                    return gcarry

                lax.fori_loop(0, _CHUNK // _LANES, group_body, 0)
                pltpu.sync_copy(o_v, out_hbm.at[pl.ds(cidx * _CHUNK, _CHUNK)])

            return carry

        lax.fori_loop(0, _CHUNKS_PER_W, chunk_body, 0)

    return sc_kernel


_sc_kernel = _make_sc_kernel()


def kernel(query, keys):
    q = query.reshape(_BITS).astype(jnp.float32)
    kflat = keys.reshape(_N_KEYS * _BITS)
    return _sc_kernel(q, kflat)


# speedup vs baseline: 1.2753x; 1.2753x over previous
import functools

import jax
import jax.numpy as jnp
from jax import lax
from jax.experimental import pallas as pl
from jax.experimental.pallas import tpu as pltpu
from jax.experimental.pallas import tpu_sc as plsc

_N_KEYS = 100000
_BITS = 512
_LANES = 16
_NVEC = _BITS // _LANES

_CHUNK = 80
_CROWS = _CHUNK // _LANES
_CBUF = _CHUNK * _BITS
_NCHUNKS = _N_KEYS // _CHUNK
_NW = 32
_ITERS = -(-_NCHUNKS // _NW)

_GATHER_DNUMS = lax.GatherDimensionNumbers(
    offset_dims=(), collapsed_slice_dims=(0,), start_index_map=(0,))


def _lane_permute(v, idx):
    return lax.gather(
        v, idx[:, None], dimension_numbers=_GATHER_DNUMS, slice_sizes=(1,),
        mode=lax.GatherScatterMode.PROMISE_IN_BOUNDS)


def _merge_tree(vecs, perms, masks):
    for k in range(4):
        nxt = []
        for t in range(len(vecs) // 2):
            a, b = vecs[2 * t], vecs[2 * t + 1]
            fa = a + _lane_permute(a, perms[k])
            fb = b + _lane_permute(b, perms[k])
            nxt.append(jnp.where(masks[k], fa, fb))
        vecs = nxt
    return vecs[0]


def _make_sc_kernel():
    mesh = plsc.VectorSubcoreMesh(core_axis_name="c", subcore_axis_name="s")

    @functools.partial(
        pl.kernel,
        mesh=mesh,
        out_type=jax.ShapeDtypeStruct((_N_KEYS,), jnp.float32),
        scratch_types=[
            pltpu.VMEM((_BITS,), jnp.float32),
            pltpu.VMEM((_BITS,), jnp.float32),
            pltpu.VMEM((_CBUF,), jnp.float32),
            pltpu.VMEM((_CBUF,), jnp.float32),
            pltpu.VMEM((_CHUNK,), jnp.float32),
            pltpu.VMEM((_CHUNK,), jnp.float32),
            pltpu.SemaphoreType.DMA,
            pltpu.SemaphoreType.DMA,
            pltpu.SemaphoreType.DMA,
            pltpu.SemaphoreType.DMA,
        ],
    )
    def sc_kernel(q_hbm, keys_hbm, out_hbm,
                  q_v, s_v, k0, k1, o0, o1, sk0, sk1, so0, so1):
        cid = lax.axis_index("c")
        sid = lax.axis_index("s")
        wid = sid * 2 + cid

        pltpu.sync_copy(q_hbm, q_v)

        lane_iota = lax.iota(jnp.int32, _LANES)
        perms = [lane_iota ^ (1 << k) for k in range(4)]
        masks = [(lane_iota & (1 << k)) == 0 for k in range(4)]

        n1_acc = jnp.zeros((_LANES,), jnp.float32)
        for i in range(_NVEC):
            qc = q_v[pl.ds(i * _LANES, _LANES)]
            s_v[pl.ds(i * _LANES, _LANES)] = 1.0 - 2.0 * qc
            n1_acc = n1_acc + qc
        n1 = n1_acc
        for k in range(4):
            n1 = n1 + _lane_permute(n1, perms[k])
        thresh = 1.0 - n1

        def cidx_of(jj):
            return jnp.minimum(wid + jj * _NW, _NCHUNKS - 1)

        bufs = ((k0, o0, sk0, so0), (k1, o1, sk1, so1))

        pltpu.async_copy(keys_hbm.at[pl.ds(cidx_of(0) * _CBUF, _CBUF)], k0, sk0)
        pltpu.async_copy(keys_hbm.at[pl.ds(cidx_of(1) * _CBUF, _CBUF)], k1, sk1)

        def iter_body(t, carry):
            j2 = t * 2
            for b in range(2):
                kb, ob, skb, sob = bufs[b]
                jj = j2 + b
                cidx = cidx_of(jj)
                pltpu.make_async_copy(
                    keys_hbm.at[pl.ds(cidx * _CBUF, _CBUF)], kb, skb).wait()

                @pl.when(jj >= 2)
                def _():
                    pltpu.make_async_copy(
                        ob, out_hbm.at[pl.ds(cidx_of(jj - 2) * _CHUNK, _CHUNK)],
                        sob).wait()

                def group_body(g, gc):
                    goff = g * (_LANES * _BITS)
                    accs = [jnp.zeros((_LANES,), jnp.float32)
                            for _ in range(_LANES)]
                    for ci in range(_NVEC):
                        ss = s_v[pl.ds(ci * _LANES, _LANES)]
                        coff = goff + ci * _LANES
                        for r in range(_LANES):
                            kk = kb[pl.ds(coff + r * _BITS, _LANES)]
                            accs[r] = accs[r] + kk * ss
                    d = _merge_tree(accs, perms, masks)
                    res = jnp.where(d <= thresh, 1.0, 0.0).astype(jnp.float32)
                    ob[pl.ds(g * _LANES, _LANES)] = res
                    return gc

                lax.fori_loop(0, _CROWS, group_body, 0)

                pltpu.async_copy(
                    ob, out_hbm.at[pl.ds(cidx * _CHUNK, _CHUNK)], sob)

                @pl.when(jj + 2 < _ITERS)
                def _():
                    pltpu.async_copy(
                        keys_hbm.at[pl.ds(cidx_of(jj + 2) * _CBUF, _CBUF)],
                        kb, skb)

            return carry

        lax.fori_loop(0, _ITERS // 2, iter_body, 0)

        for b in range(2):
            kb, ob, skb, sob = bufs[b]
            pltpu.make_async_copy(
                ob,
                out_hbm.at[pl.ds(cidx_of(_ITERS - 2 + b) * _CHUNK, _CHUNK)],
                sob).wait()

    return sc_kernel


_sc_kernel = _make_sc_kernel()


def kernel(query, keys):
    q = query.reshape(_BITS).astype(jnp.float32)
    kflat = keys.reshape(_N_KEYS * _BITS)
    return _sc_kernel(q, kflat)


# speedup vs baseline: 1.5346x; 1.2033x over previous
import functools

import jax
import jax.numpy as jnp
from jax import lax
from jax.experimental import pallas as pl
from jax.experimental.pallas import tpu as pltpu
from jax.experimental.pallas import tpu_sc as plsc

_N_KEYS = 100000
_BITS = 512
_LANES = 16
_NVEC = _BITS // _LANES

_TC_BLK = 2000
_N_SC = 56000
_N_TC = _N_KEYS - _N_SC

_CHUNK = 80
_CROWS = _CHUNK // _LANES
_CBUF = _CHUNK * _BITS
_NCHUNKS = _N_SC // _CHUNK
_NW = 32
_ITERS = 2 * (-(-_NCHUNKS // (2 * _NW)))

_GATHER_DNUMS = lax.GatherDimensionNumbers(
    offset_dims=(), collapsed_slice_dims=(0,), start_index_map=(0,))


def _lane_permute(v, idx):
    return lax.gather(
        v, idx[:, None], dimension_numbers=_GATHER_DNUMS, slice_sizes=(1,),
        mode=lax.GatherScatterMode.PROMISE_IN_BOUNDS)


def _merge_tree(vecs, perms, masks):
    for k in range(4):
        nxt = []
        for t in range(len(vecs) // 2):
            a, b = vecs[2 * t], vecs[2 * t + 1]
            fa = a + _lane_permute(a, perms[k])
            fb = b + _lane_permute(b, perms[k])
            nxt.append(jnp.where(masks[k], fa, fb))
        vecs = nxt
    return vecs[0]


def _make_sc_kernel():
    mesh = plsc.VectorSubcoreMesh(core_axis_name="c", subcore_axis_name="s")

    @functools.partial(
        pl.kernel,
        mesh=mesh,
        out_type=jax.ShapeDtypeStruct((_N_SC,), jnp.float32),
        scratch_types=[
            pltpu.VMEM((_BITS,), jnp.float32),
            pltpu.VMEM((_BITS,), jnp.float32),
            pltpu.VMEM((_CBUF,), jnp.float32),
            pltpu.VMEM((_CBUF,), jnp.float32),
            pltpu.VMEM((_CHUNK,), jnp.float32),
            pltpu.VMEM((_CHUNK,), jnp.float32),
            pltpu.SemaphoreType.DMA,
            pltpu.SemaphoreType.DMA,
            pltpu.SemaphoreType.DMA,
            pltpu.SemaphoreType.DMA,
        ],
    )
    def sc_kernel(q_hbm, keys_hbm, out_hbm,
                  q_v, s_v, k0, k1, o0, o1, sk0, sk1, so0, so1):
        cid = lax.axis_index("c")
        sid = lax.axis_index("s")
        wid = sid * 2 + cid

        pltpu.sync_copy(q_hbm, q_v)

        lane_iota = lax.iota(jnp.int32, _LANES)
        perms = [lane_iota ^ (1 << k) for k in range(4)]
        masks = [(lane_iota & (1 << k)) == 0 for k in range(4)]

        n1_acc = jnp.zeros((_LANES,), jnp.float32)
        for i in range(_NVEC):
            qc = q_v[pl.ds(i * _LANES, _LANES)]
            s_v[pl.ds(i * _LANES, _LANES)] = 1.0 - 2.0 * qc
            n1_acc = n1_acc + qc
        n1 = n1_acc
        for k in range(4):
            n1 = n1 + _lane_permute(n1, perms[k])
        thresh = 1.0 - n1

        def cidx_of(jj):
            return jnp.minimum(wid + jj * _NW, _NCHUNKS - 1)

        bufs = ((k0, o0, sk0, so0), (k1, o1, sk1, so1))

        pltpu.async_copy(keys_hbm.at[pl.ds(cidx_of(0) * _CBUF, _CBUF)], k0, sk0)
        pltpu.async_copy(keys_hbm.at[pl.ds(cidx_of(1) * _CBUF, _CBUF)], k1, sk1)

        def iter_body(t, carry):
            j2 = t * 2
            for b in range(2):
                kb, ob, skb, sob = bufs[b]
                jj = j2 + b
                cidx = cidx_of(jj)
                pltpu.make_async_copy(
                    keys_hbm.at[pl.ds(cidx * _CBUF, _CBUF)], kb, skb).wait()

                @pl.when(jj >= 2)
                def _():
                    pltpu.make_async_copy(
                        ob, out_hbm.at[pl.ds(cidx_of(jj - 2) * _CHUNK, _CHUNK)],
                        sob).wait()

                def group_body(g, gc):
                    goff = g * (_LANES * _BITS)
                    accs = [jnp.zeros((_LANES,), jnp.float32)
                            for _ in range(_LANES)]
                    for ci in range(_NVEC):
                        ss = s_v[pl.ds(ci * _LANES, _LANES)]
                        coff = goff + ci * _LANES
                        for r in range(_LANES):
                            kk = kb[pl.ds(coff + r * _BITS, _LANES)]
                            accs[r] = accs[r] + kk * ss
                    d = _merge_tree(accs, perms, masks)
                    res = jnp.where(d <= thresh, 1.0, 0.0).astype(jnp.float32)
                    ob[pl.ds(g * _LANES, _LANES)] = res
                    return gc

                lax.fori_loop(0, _CROWS, group_body, 0)

                pltpu.async_copy(
                    ob, out_hbm.at[pl.ds(cidx * _CHUNK, _CHUNK)], sob)

                @pl.when(jj + 2 < _ITERS)
                def _():
                    pltpu.async_copy(
                        keys_hbm.at[pl.ds(cidx_of(jj + 2) * _CBUF, _CBUF)],
                        kb, skb)

            return carry

        lax.fori_loop(0, _ITERS // 2, iter_body, 0)

        for b in range(2):
            kb, ob, skb, sob = bufs[b]
            pltpu.make_async_copy(
                ob,
                out_hbm.at[pl.ds(cidx_of(_ITERS - 2 + b) * _CHUNK, _CHUNK)],
                sob).wait()

    return sc_kernel


_sc_kernel = _make_sc_kernel()

_TC_OFF = _N_SC // _TC_BLK
_TC_GRID = _N_TC // _TC_BLK


def _tc_body(q_ref, k_ref, o_ref):
    q = q_ref[0, :]
    s = 1.0 - 2.0 * q
    n1 = jnp.sum(q)
    k = k_ref[0]
    dist = n1 + jnp.sum(k * s[None, :], axis=1)
    o_ref[0, 0, :] = jnp.where(dist <= 1.0, 1.0, 0.0).astype(jnp.float32)


def _tc_part(q2, k3):
    return pl.pallas_call(
        _tc_body,
        grid=(_TC_GRID,),
        in_specs=[
            pl.BlockSpec((1, _BITS), lambda i: (0, 0)),
            pl.BlockSpec((1, _TC_BLK, _BITS), lambda i: (i + _TC_OFF, 0, 0)),
        ],
        out_specs=pl.BlockSpec((1, 1, _TC_BLK), lambda i: (i, 0, 0)),
        out_shape=jax.ShapeDtypeStruct((_TC_GRID, 1, _TC_BLK), jnp.float32),
    )(q2, k3)


def kernel(query, keys):
    q = query.reshape(_BITS).astype(jnp.float32)
    kflat = keys.reshape(_N_KEYS * _BITS)
    out_sc = _sc_kernel(q, kflat)
    out_tc = _tc_part(q.reshape(1, _BITS),
                      keys.reshape(_N_KEYS // _TC_BLK, _TC_BLK, _BITS))
    return jnp.concatenate([out_sc, out_tc.reshape(_N_TC)])


# speedup vs baseline: 4.2584x; 2.7750x over previous
import functools

import jax
import jax.numpy as jnp
from jax import lax
from jax.experimental import pallas as pl
from jax.experimental.pallas import tpu as pltpu
from jax.experimental.pallas import tpu_sc as plsc

_N_KEYS = 100000
_BITS = 512
_LANES = 16
_NVEC = _BITS // _LANES

_TC_BLK = 2000
_N_SC = 56000
_N_TC = _N_KEYS - _N_SC

_CHUNK = 80
_CROWS = _CHUNK // _LANES
_CBUF = _CHUNK * _BITS
_NCHUNKS = _N_SC // _CHUNK
_NW = 32
_ITERS = 2 * (-(-_NCHUNKS // (2 * _NW)))

_GATHER_DNUMS = lax.GatherDimensionNumbers(
    offset_dims=(), collapsed_slice_dims=(0,), start_index_map=(0,))


def _lane_permute(v, idx):
    return lax.gather(
        v, idx[:, None], dimension_numbers=_GATHER_DNUMS, slice_sizes=(1,),
        mode=lax.GatherScatterMode.PROMISE_IN_BOUNDS)


def _merge_tree(vecs, perms, masks):
    for k in range(4):
        nxt = []
        for t in range(len(vecs) // 2):
            a, b = vecs[2 * t], vecs[2 * t + 1]
            fa = a + _lane_permute(a, perms[k])
            fb = b + _lane_permute(b, perms[k])
            nxt.append(jnp.where(masks[k], fa, fb))
        vecs = nxt
    return vecs[0]


def _make_sc_kernel():
    mesh = plsc.VectorSubcoreMesh(core_axis_name="c", subcore_axis_name="s")

    @functools.partial(
        pl.kernel,
        mesh=mesh,
        out_type=jax.ShapeDtypeStruct((_N_SC,), jnp.float32),
        scratch_types=[
            pltpu.VMEM((_BITS,), jnp.float32),
            pltpu.VMEM((_BITS,), jnp.float32),
            pltpu.VMEM((_CHUNK, _BITS), jnp.float32),
            pltpu.VMEM((_CHUNK, _BITS), jnp.float32),
            pltpu.VMEM((_CHUNK,), jnp.float32),
            pltpu.VMEM((_CHUNK,), jnp.float32),
            pltpu.SemaphoreType.DMA,
            pltpu.SemaphoreType.DMA,
            pltpu.SemaphoreType.DMA,
            pltpu.SemaphoreType.DMA,
        ],
    )
    def sc_kernel(q_hbm, keys_hbm, out_hbm,
                  q_v, s_v, k0, k1, o0, o1, sk0, sk1, so0, so1):
        cid = lax.axis_index("c")
        sid = lax.axis_index("s")
        wid = sid * 2 + cid

        pltpu.sync_copy(q_hbm, q_v)

        lane_iota = lax.iota(jnp.int32, _LANES)
        perms = [lane_iota ^ (1 << k) for k in range(4)]
        masks = [(lane_iota & (1 << k)) == 0 for k in range(4)]

        n1_acc = jnp.zeros((_LANES,), jnp.float32)
        for i in range(_NVEC):
            qc = q_v[pl.ds(i * _LANES, _LANES)]
            s_v[pl.ds(i * _LANES, _LANES)] = 1.0 - 2.0 * qc
            n1_acc = n1_acc + qc
        n1 = n1_acc
        for k in range(4):
            n1 = n1 + _lane_permute(n1, perms[k])
        thresh = 1.0 - n1

        def cidx_of(jj):
            return jnp.minimum(wid + jj * _NW, _NCHUNKS - 1)

        bufs = ((k0, o0, sk0, so0), (k1, o1, sk1, so1))

        pltpu.async_copy(
            keys_hbm.at[pl.ds(cidx_of(0) * _CHUNK, _CHUNK), :], k0, sk0)
        pltpu.async_copy(
            keys_hbm.at[pl.ds(cidx_of(1) * _CHUNK, _CHUNK), :], k1, sk1)

        def iter_body(t, carry):
            j2 = t * 2
            for b in range(2):
                kb, ob, skb, sob = bufs[b]
                jj = j2 + b
                cidx = cidx_of(jj)
                pltpu.make_async_copy(
                    keys_hbm.at[pl.ds(cidx * _CHUNK, _CHUNK), :], kb,
                    skb).wait()

                @pl.when(jj >= 2)
                def _():
                    pltpu.make_async_copy(
                        ob, out_hbm.at[pl.ds(cidx_of(jj - 2) * _CHUNK, _CHUNK)],
                        sob).wait()

                def group_body(g, gc):
                    grow = g * _LANES

                    def ci_body(ci, accs):
                        cb = ci * _LANES
                        ss = s_v[pl.ds(cb, _LANES)]
                        return tuple(
                            accs[r] + kb[grow + r, pl.ds(cb, _LANES)] * ss
                            for r in range(_LANES))

                    accs = lax.fori_loop(
                        0, _NVEC, ci_body,
                        tuple(jnp.zeros((_LANES,), jnp.float32)
                              for _ in range(_LANES)),
                        unroll=2)
                    d = _merge_tree(list(accs), perms, masks)
                    res = jnp.where(d <= thresh, 1.0, 0.0).astype(jnp.float32)
                    ob[pl.ds(g * _LANES, _LANES)] = res
                    return gc

                lax.fori_loop(0, _CROWS, group_body, 0)

                pltpu.async_copy(
                    ob, out_hbm.at[pl.ds(cidx * _CHUNK, _CHUNK)], sob)

                @pl.when(jj + 2 < _ITERS)
                def _():
                    pltpu.async_copy(
                        keys_hbm.at[pl.ds(cidx_of(jj + 2) * _CHUNK, _CHUNK), :],
                        kb, skb)

            return carry

        lax.fori_loop(0, _ITERS // 2, iter_body, 0)

        for b in range(2):
            kb, ob, skb, sob = bufs[b]
            pltpu.make_async_copy(
                ob,
                out_hbm.at[pl.ds(cidx_of(_ITERS - 2 + b) * _CHUNK, _CHUNK)],
                sob).wait()

    return sc_kernel


_sc_kernel = _make_sc_kernel()

_TC_OFF = _N_SC // _TC_BLK
_TC_GRID = _N_TC // _TC_BLK


def _tc_body(q_ref, k_ref, o_ref):
    q = q_ref[0, :]
    s = 1.0 - 2.0 * q
    n1 = jnp.sum(q)
    k = k_ref[0]
    dist = n1 + jnp.sum(k * s[None, :], axis=1)
    o_ref[0, 0, :] = jnp.where(dist <= 1.0, 1.0, 0.0).astype(jnp.float32)


def _tc_part(q2, k3):
    return pl.pallas_call(
        _tc_body,
        grid=(_TC_GRID,),
        in_specs=[
            pl.BlockSpec((1, _BITS), lambda i: (0, 0)),
            pl.BlockSpec((1, _TC_BLK, _BITS), lambda i: (i + _TC_OFF, 0, 0)),
        ],
        out_specs=pl.BlockSpec((1, 1, _TC_BLK), lambda i: (i, 0, 0)),
        out_shape=jax.ShapeDtypeStruct((_TC_GRID, 1, _TC_BLK), jnp.float32),
    )(q2, k3)


def kernel(query, keys):
    q = query.reshape(_BITS).astype(jnp.float32)
    out_sc = _sc_kernel(q, keys)
    out_tc = _tc_part(q.reshape(1, _BITS),
                      keys.reshape(_N_KEYS // _TC_BLK, _TC_BLK, _BITS))
    return jnp.concatenate([out_sc, out_tc.reshape(_N_TC)])
